# baseline (speedup 1.0000x reference)
import functools

import jax
import jax.numpy as jnp
from jax import lax
from jax.experimental import pallas as pl
from jax.experimental.pallas import tpu as pltpu
from jax.experimental.pallas import tpu_sc as plsc

_BL_TC = 2048
_L_SC = 2048
_NC, _NS = 2, 16
_NW = _NC * _NS
_CHUNK = 64


def _vperm(x, idx):
    dn = lax.GatherDimensionNumbers(
        offset_dims=(), collapsed_slice_dims=(0,), start_index_map=(0,))
    return lax.gather(x, idx[:, None], dn, (1,),
                      mode=lax.GatherScatterMode.PROMISE_IN_BOUNDS)


_TC_CH = 1024
_TC_NBUF = 6


def _tc_kernel(B, L, l_tc, x_ref, m_ref, w_ref, b_ref, t_ref, o_ref,
               bufs, sems):
    D = w_ref.shape[0]
    ch = _TC_CH
    nbuf = _TC_NBUF
    cpb = l_tc // ch
    nch = B * cpb
    w = w_ref[...]
    bb = b_ref[0, 0]

    def src_row(c):
        return (c // cpb) * L + (c % cpb) * ch

    def copy(c):
        return pltpu.make_async_copy(
            x_ref.at[pl.ds(src_row(c), ch)], bufs.at[c % nbuf], sems.at[c % nbuf])

    for c in range(min(nbuf - 1, nch)):
        copy(c).start()
    for c in range(nch):
        if c + nbuf - 1 < nch:
            copy(c + nbuf - 1).start()
        copy(c).wait()
        r = jnp.dot(bufs[c % nbuf], w, preferred_element_type=jnp.float32)
        pos = (c % cpb) * ch
        dst = (c // cpb) * l_tc + pos
        o_ref[pl.ds(dst, ch)] = (
            (r + bb) * m_ref[pl.ds(src_row(c), ch)] * jnp.exp(t_ref[pl.ds(pos, ch)]))


def _tc_part(encoded_parents, masks, W, b, log_site_table, l_tc):
    B, L, D = encoded_parents.shape
    x2 = encoded_parents.reshape(B * L, D)
    m2 = masks.reshape(B * L, 1)
    b2 = b.reshape(1, 1)
    out = pl.pallas_call(
        functools.partial(_tc_kernel, B, L, l_tc),
        in_specs=[
            pl.BlockSpec(memory_space=pl.ANY),
            pl.BlockSpec(memory_space=pltpu.MemorySpace.VMEM),
            pl.BlockSpec(memory_space=pltpu.MemorySpace.VMEM),
            pl.BlockSpec(memory_space=pltpu.MemorySpace.VMEM),
            pl.BlockSpec(memory_space=pltpu.MemorySpace.VMEM),
        ],
        out_specs=pl.BlockSpec(memory_space=pltpu.MemorySpace.VMEM),
        out_shape=jax.ShapeDtypeStruct((B * l_tc, 1), jnp.float32),
        scratch_shapes=[
            pltpu.VMEM((_TC_NBUF, _TC_CH, D), jnp.float32),
            pltpu.SemaphoreType.DMA((_TC_NBUF,)),
        ],
    )(x2, m2, W, b2, log_site_table)
    return out.reshape(B, l_tc)


def _sc_body(B, L, D, l_tc,
             x_hbm, m_hbm, t_hbm, w_hbm, b_hbm, o_hbm,
             wv, bv, tv, mv, ov, xv0, xv1, sem0, sem1):
    l_per_w = _L_SC // _NW
    wid = lax.axis_index("s") * _NC + lax.axis_index("c")
    pos0 = l_tc + wid * l_per_w
    nk = D // 16

    pltpu.sync_copy(w_hbm, wv)
    pltpu.sync_copy(b_hbm, bv)
    lane = lax.broadcasted_iota(jnp.int32, (16,), 0)
    perms = [lane ^ (1 << s) for s in range(4)]

    chunks_per_batch = l_per_w // _CHUNK
    n_chunks = B * chunks_per_batch
    bufs = (xv0, xv1)
    sems = (sem0, sem1)

    def chunk_row0(c):
        bb = c // chunks_per_batch
        off = (c % chunks_per_batch) * _CHUNK
        return bb * L + pos0 + off, bb, off

    r0, _, _ = chunk_row0(0)
    cp = pltpu.make_async_copy(x_hbm.at[pl.ds(r0, _CHUNK)], bufs[0], sems[0])
    cp.start()

    for c in range(n_chunks):
        row0, bb, off = chunk_row0(c)
        if c + 1 < n_chunks:
            r1, _, _ = chunk_row0(c + 1)
            nxt = pltpu.make_async_copy(
                x_hbm.at[pl.ds(r1, _CHUNK)], bufs[(c + 1) % 2], sems[(c + 1) % 2])
            nxt.start()
        pltpu.make_async_copy(
            x_hbm.at[pl.ds(row0, _CHUNK)], bufs[c % 2], sems[c % 2]).wait()
        xv = bufs[c % 2]

        pltpu.sync_copy(m_hbm.at[pl.ds(row0, _CHUNK)], mv)
        pltpu.sync_copy(t_hbm.at[pl.ds(pos0 + off, _CHUNK)], tv)

        def grp_body(g, carry, xv=xv):
            def row_body(i, outv, xv=xv):
                r = g * 16 + i
                naccs = 8
                accs = [xv[r, pl.ds(k * 16, 16)] * wv[pl.ds(k * 16, 16)]
                        for k in range(naccs)]
                for k in range(naccs, nk):
                    j = k % naccs
                    accs[j] = (accs[j]
                               + xv[r, pl.ds(k * 16, 16)] * wv[pl.ds(k * 16, 16)])
                while len(accs) > 1:
                    accs = [accs[j] + accs[j + 1]
                            for j in range(0, len(accs), 2)]
                acc = accs[0]
                for p in perms:
                    acc = acc + _vperm(acc, p)
                return jnp.where(lane == i, acc, outv)

            outv = lax.fori_loop(0, 16, row_body, jnp.zeros((16,), jnp.float32))
            gs = pl.ds(g * 16, 16)
            ov[gs] = (outv + bv[...]) * mv[gs] * jnp.exp(tv[gs])
            return carry

        lax.fori_loop(0, _CHUNK // 16, grp_body, 0)

        pltpu.sync_copy(ov, o_hbm.at[pl.ds(bb * _L_SC + (pos0 - l_tc) + off, _CHUNK)])


def _sc_part(encoded_parents, masks, W, b, log_site_table, l_tc):
    B, L, D = encoded_parents.shape
    x2 = encoded_parents.reshape(B * L, D)
    m2 = masks.reshape(B * L)
    t1 = log_site_table.reshape(-1)[:L]
    w1 = W.reshape(D)
    b16 = jnp.broadcast_to(b.reshape(1), (16,))

    mesh = plsc.VectorSubcoreMesh(core_axis_name="c", subcore_axis_name="s")
    run = functools.partial(
        pl.kernel,
        out_type=jax.ShapeDtypeStruct((B * _L_SC,), jnp.float32),
        mesh=mesh,
        scratch_types=[
            pltpu.VMEM((D,), jnp.float32),
            pltpu.VMEM((16,), jnp.float32),
            pltpu.VMEM((_CHUNK,), jnp.float32),
            pltpu.VMEM((_CHUNK,), jnp.float32),
            pltpu.VMEM((_CHUNK,), jnp.float32),
            pltpu.VMEM((_CHUNK, D), jnp.float32),
            pltpu.VMEM((_CHUNK, D), jnp.float32),
            pltpu.SemaphoreType.DMA,
            pltpu.SemaphoreType.DMA,
        ],
    )(functools.partial(_sc_body, B, L, D, l_tc))
    o = run(x2, m2, t1, w1, b16)
    return o.reshape(B, _L_SC)


def kernel(encoded_parents, masks, W, b, log_site_table):
    B, L, D = encoded_parents.shape
    l_tc = L - _L_SC
    o_sc = _sc_part(encoded_parents, masks, W, b, log_site_table, l_tc)
    o_tc = _tc_part(encoded_parents, masks, W, b, log_site_table, l_tc)
    return jnp.concatenate([o_tc, o_sc], axis=1)


# speedup vs baseline: 1.1804x; 1.1804x over previous
import functools

import jax
import jax.numpy as jnp
from jax import lax
from jax.experimental import pallas as pl
from jax.experimental.pallas import tpu as pltpu
from jax.experimental.pallas import tpu_sc as plsc

_BL_TC = 2048
_L_SC = 2048
_NC, _NS = 2, 16
_NW = _NC * _NS
_CHUNK = 64


def _vperm(x, idx):
    dn = lax.GatherDimensionNumbers(
        offset_dims=(), collapsed_slice_dims=(0,), start_index_map=(0,))
    return lax.gather(x, idx[:, None], dn, (1,),
                      mode=lax.GatherScatterMode.PROMISE_IN_BOUNDS)


_TC_CH = 1024
_TC_NBUF = 6


def _tc_kernel(B, L, l_tc, x_ref, m_ref, w_ref, b_ref, t_ref, o_ref,
               bufs, sems):
    D = w_ref.shape[0]
    ch = _TC_CH
    nbuf = _TC_NBUF
    cpb = l_tc // ch
    nch = B * cpb
    w = w_ref[...]
    bb = b_ref[0, 0]

    def src_row(c):
        return (c // cpb) * L + (c % cpb) * ch

    def copy(c):
        return pltpu.make_async_copy(
            x_ref.at[pl.ds(src_row(c), ch)], bufs.at[c % nbuf], sems.at[c % nbuf])

    for c in range(min(nbuf - 1, nch)):
        copy(c).start()
    for c in range(nch):
        if c + nbuf - 1 < nch:
            copy(c + nbuf - 1).start()
        copy(c).wait()
        r = jnp.dot(bufs[c % nbuf], w, preferred_element_type=jnp.float32)
        pos = (c % cpb) * ch
        dst = (c // cpb) * l_tc + pos
        o_ref[pl.ds(dst, ch)] = (
            (r + bb) * m_ref[pl.ds(src_row(c), ch)] * jnp.exp(t_ref[pl.ds(pos, ch)]))


def _tc_part(encoded_parents, masks, W, b, log_site_table, l_tc):
    B, L, D = encoded_parents.shape
    x2 = encoded_parents.reshape(B * L, D)
    m2 = masks.reshape(B * L, 1)
    b2 = b.reshape(1, 1)
    out = pl.pallas_call(
        functools.partial(_tc_kernel, B, L, l_tc),
        in_specs=[
            pl.BlockSpec(memory_space=pl.ANY),
            pl.BlockSpec(memory_space=pltpu.MemorySpace.VMEM),
            pl.BlockSpec(memory_space=pltpu.MemorySpace.VMEM),
            pl.BlockSpec(memory_space=pltpu.MemorySpace.VMEM),
            pl.BlockSpec(memory_space=pltpu.MemorySpace.VMEM),
        ],
        out_specs=pl.BlockSpec(memory_space=pltpu.MemorySpace.VMEM),
        out_shape=jax.ShapeDtypeStruct((B * l_tc, 1), jnp.float32),
        scratch_shapes=[
            pltpu.VMEM((_TC_NBUF, _TC_CH, D), jnp.float32),
            pltpu.SemaphoreType.DMA((_TC_NBUF,)),
        ],
    )(x2, m2, W, b2, log_site_table)
    return out.reshape(B, l_tc)


def _sc_body(B, L, D, l_tc,
             x_hbm, m_hbm, t_hbm, w_hbm, b_hbm, o_hbm,
             wv, bv, tv, mv, ov, xv0, xv1, sem0, sem1):
    l_per_w = _L_SC // _NW
    wid = lax.axis_index("s") * _NC + lax.axis_index("c")
    pos0 = l_tc + wid * l_per_w
    nk = D // 16

    pltpu.sync_copy(w_hbm, wv)
    pltpu.sync_copy(b_hbm, bv)
    lane = lax.broadcasted_iota(jnp.int32, (16,), 0)
    perms = [lane ^ (1 << s) for s in range(4)]

    chunks_per_batch = l_per_w // _CHUNK
    n_chunks = B * chunks_per_batch
    bufs = (xv0, xv1)
    sems = (sem0, sem1)

    def chunk_row0(c):
        bb = c // chunks_per_batch
        off = (c % chunks_per_batch) * _CHUNK
        return bb * L + pos0 + off, bb, off

    r0, _, _ = chunk_row0(0)
    cp = pltpu.make_async_copy(x_hbm.at[pl.ds(r0, _CHUNK)], bufs[0], sems[0])
    cp.start()

    for c in range(n_chunks):
        row0, bb, off = chunk_row0(c)
        if c + 1 < n_chunks:
            r1, _, _ = chunk_row0(c + 1)
            nxt = pltpu.make_async_copy(
                x_hbm.at[pl.ds(r1, _CHUNK)], bufs[(c + 1) % 2], sems[(c + 1) % 2])
            nxt.start()
        pltpu.make_async_copy(
            x_hbm.at[pl.ds(row0, _CHUNK)], bufs[c % 2], sems[c % 2]).wait()
        xv = bufs[c % 2]

        pltpu.sync_copy(m_hbm.at[pl.ds(row0, _CHUNK)], mv)
        pltpu.sync_copy(t_hbm.at[pl.ds(pos0 + off, _CHUNK)], tv)

        def grp_body(g, carry, xv=xv):
            def row_body(i, outv, xv=xv):
                r = g * 16 + i
                naccs = 8
                accs = [xv[r, pl.ds(k * 16, 16)] * wv[pl.ds(k * 16, 16)]
                        for k in range(naccs)]
                for k in range(naccs, nk):
                    j = k % naccs
                    accs[j] = (accs[j]
                               + xv[r, pl.ds(k * 16, 16)] * wv[pl.ds(k * 16, 16)])
                while len(accs) > 1:
                    accs = [accs[j] + accs[j + 1]
                            for j in range(0, len(accs), 2)]
                acc = accs[0]
                for p in perms:
                    acc = acc + _vperm(acc, p)
                return jnp.where(lane == i, acc, outv)

            outv = lax.fori_loop(0, 16, row_body, jnp.zeros((16,), jnp.float32))
            gs = pl.ds(g * 16, 16)
            ov[gs] = (outv + bv[...]) * mv[gs] * jnp.exp(tv[gs])
            return carry

        lax.fori_loop(0, _CHUNK // 16, grp_body, 0)

        pltpu.sync_copy(ov, o_hbm.at[pl.ds(bb * _L_SC + (pos0 - l_tc) + off, _CHUNK)])


def _sc_part(encoded_parents, masks, W, b, log_site_table, l_tc):
    B, L, D = encoded_parents.shape
    x2 = encoded_parents.reshape(B * L, D)
    m2 = masks.reshape(B * L)
    t1 = log_site_table.reshape(-1)[:L]
    w1 = W.reshape(D)
    b16 = jnp.broadcast_to(b.reshape(1), (16,))

    mesh = plsc.VectorSubcoreMesh(core_axis_name="c", subcore_axis_name="s")
    run = functools.partial(
        pl.kernel,
        out_type=jax.ShapeDtypeStruct((B * _L_SC,), jnp.float32),
        mesh=mesh,
        scratch_types=[
            pltpu.VMEM((D,), jnp.float32),
            pltpu.VMEM((16,), jnp.float32),
            pltpu.VMEM((_CHUNK,), jnp.float32),
            pltpu.VMEM((_CHUNK,), jnp.float32),
            pltpu.VMEM((_CHUNK,), jnp.float32),
            pltpu.VMEM((_CHUNK, D), jnp.float32),
            pltpu.VMEM((_CHUNK, D), jnp.float32),
            pltpu.SemaphoreType.DMA,
            pltpu.SemaphoreType.DMA,
        ],
    )(functools.partial(_sc_body, B, L, D, l_tc))
    o = run(x2, m2, t1, w1, b16)
    return o.reshape(B, _L_SC)


def kernel(encoded_parents, masks, W, b, log_site_table):
    B, L, D = encoded_parents.shape
    l_tc = L
    o_tc = _tc_part(encoded_parents, masks, W, b, log_site_table, l_tc)
    return o_tc
